# baseline (speedup 1.0000x reference)
import jax
import jax.numpy as jnp
from jax import lax
from jax.experimental import pallas as pl
from jax.experimental.pallas import tpu as pltpu
from jax.experimental.pallas import tpu_sc as plsc

L = 16
NT = 16
CHUNK = 6272
NPAD = NT * CHUNK
NV = CHUNK // L
N_IN = 100000
TAIL = N_IN - (NT - 1) * CHUNK
PR = 128

TOTAL_NORM = 60000

NB = 256
NB4 = 128
NSLOT = 4

_mesh = plsc.VectorSubcoreMesh(
    core_axis_name="c", subcore_axis_name="s", num_cores=1)


def _select_bin(hist_ref, nbins, k):
  def total_body(c, acc):
    return acc + jnp.sum(hist_ref[pl.ds(c * L, L)])

  total = lax.fori_loop(0, nbins // L, total_body, jnp.int32(0))
  m = total - k

  lane = lax.iota(jnp.int32, L)

  def body(c, carry):
    run, bsel, psel = carry
    h = hist_ref[pl.ds(c * L, L)]
    p_incl = plsc.cumsum(h) + run
    p_excl = p_incl - h
    m_vec = jnp.full((L,), m, jnp.int32)
    cond = (p_excl <= m_vec) & (m_vec < p_incl)
    zero = jnp.zeros((L,), jnp.int32)
    bsel = bsel + jnp.sum(jnp.where(cond, lane + c * L, zero))
    psel = psel + jnp.sum(jnp.where(cond, p_incl, zero))
    run = run + jnp.sum(h)
    return run, bsel, psel

  _, bsel, psel = lax.fori_loop(
      0, nbins // L, body, (jnp.int32(0), jnp.int32(0), jnp.int32(0)))
  k_rem = psel - m
  return bsel, k_rem


def _zero_hist(hist_ref, nwords):
  zero = jnp.zeros((L,), jnp.int32)

  def body(c, _):
    hist_ref[pl.ds(c * L, L)] = zero
    return 0

  lax.fori_loop(0, nwords // L, body, 0)


def _fold_slots(hist_ref, nbins):
  def body(c, _):
    acc = hist_ref[pl.ds(c * L, L)]
    for u in range(1, NSLOT):
      acc = acc + hist_ref[pl.ds(u * NB + c * L, L)]
    hist_ref[pl.ds(c * L, L)] = acc
    return 0

  lax.fori_loop(0, nbins // L, body, 0)


def _hist_pass(key_ref, hist_ref, match_fn, bin_fn, nbins):
  _zero_hist(hist_ref, NSLOT * NB)

  @plsc.parallel_loop(0, NV, unroll=NSLOT)
  def _(i):
    key = key_ref[pl.ds(i * L, L)]
    match = match_fn(key)
    bins = bin_fn(key) + ((i & (NSLOT - 1)) << 8)
    counts, last = plsc.scan_count(bins, mask=match)
    plsc.addupdate_scatter(hist_ref, [bins], counts, mask=last)

  _fold_slots(hist_ref, nbins)


def _merge_staged(sh_hist, merge_v, hist_v, nbins):
  pltpu.sync_copy(sh_hist, merge_v)

  def body(c, _):
    acc = jnp.zeros((L,), jnp.int32)
    for t in range(NT):
      acc = acc + merge_v[pl.ds(t * NB + c * L, L)]
    hist_v[pl.ds(c * L, L)] = acc
    return 0

  lax.fori_loop(0, nbins // L, body, 0)


def _bcast_write(bc_v, sh_bcast, vec):
  bc_v[pl.ds(0, L)] = vec
  pltpu.sync_copy(bc_v, sh_bcast)


def _sc_body(x_hbm, tgt_hbm, out_hbm,
             xy_v, tgt_v, key_v, hist_v, merge_v,
             part_v, stage_v, bc_v, tmp_f, smem_i, smem_f, sem,
             sh_hist, sh_part, sh_fsum, sh_bcast):
  wid = lax.axis_index("s")
  base = wid * CHUNK

  @pl.when(wid < NT - 1)
  def _():
    c0 = pltpu.make_async_copy(
        x_hbm.at[pl.ds(base * 2, CHUNK * 2)], xy_v, sem)
    c1 = pltpu.make_async_copy(tgt_hbm.at[pl.ds(base, CHUNK)], tgt_v, sem)
    c0.start()
    c1.start()
    c0.wait()
    c1.wait()

  @pl.when(wid == NT - 1)
  def _():
    c0 = pltpu.make_async_copy(
        x_hbm.at[pl.ds(base * 2, TAIL * 2)], xy_v.at[pl.ds(0, TAIL * 2)], sem)
    c1 = pltpu.make_async_copy(
        tgt_hbm.at[pl.ds(base, TAIL)], tgt_v.at[pl.ds(0, TAIL)], sem)
    c0.start()
    c1.start()
    c0.wait()
    c1.wait()
    pad_t = jnp.full((L,), 2, jnp.int32)

    def fill(c, _):
      tgt_v[pl.ds(TAIL + c * L, L)] = pad_t
      return 0

    lax.fori_loop(0, (CHUNK - TAIL) // L, fill, 0)

  _zero_hist(hist_v, NSLOT * NB)

  ones_i = jnp.ones((L,), jnp.int32)
  zeros_f = jnp.zeros((L,), jnp.float32)
  zeros_i = jnp.zeros((L,), jnp.int32)
  lane2 = lax.iota(jnp.int32, L) * 2

  @plsc.parallel_loop(0, NV, unroll=NSLOT,
                      carry=(zeros_f, zeros_i, zeros_i))
  def phase_a(i, carry):
    pos_acc, npos_acc, nneg_acc = carry
    sl = pl.ds(i * L, L)
    rows = i * (2 * L) + lane2
    a0 = plsc.load_gather(xy_v, [rows])
    a1 = plsc.load_gather(xy_v, [rows + 1])
    t = tgt_v[sl]
    d = a0 - a1
    ad = jnp.abs(d)
    e = jnp.exp(-ad)
    s = e / (2.0 + e)
    s2 = s * s
    p = 1.0 + s2 * (0.33333334 + s2 * (0.2 + s2 * (0.14285715 + s2 * 0.11111111)))
    l1p = 2.0 * s * p
    ce_pos = jnp.maximum(d, 0.0) + l1p
    ce_neg = jnp.maximum(-d, 0.0) + l1p
    is_pos = t == 1
    is_neg = t == 0
    pos_acc = pos_acc + jnp.where(is_pos, ce_pos, zeros_f)
    npos_acc = npos_acc + jnp.where(is_pos, ones_i, zeros_i)
    nneg_acc = nneg_acc + jnp.where(is_neg, ones_i, zeros_i)
    key = jnp.where(is_neg, lax.bitcast_convert_type(ce_neg, jnp.int32), -1)
    key_v[sl] = key
    bins = lax.shift_right_arithmetic(key, 23) + ((i & (NSLOT - 1)) << 8)
    counts, last = plsc.scan_count(bins, mask=is_neg)
    plsc.addupdate_scatter(hist_v, [bins], counts, mask=last)
    return pos_acc, npos_acc, nneg_acc

  pos_acc, npos_acc, nneg_acc = phase_a
  _fold_slots(hist_v, NB)

  part_v[pl.ds(0, L)] = lax.bitcast_convert_type(pos_acc, jnp.int32)
  part_v[pl.ds(L, L)] = npos_acc
  part_v[pl.ds(2 * L, L)] = nneg_acc
  pltpu.sync_copy(part_v, sh_part.at[pl.ds(wid * PR, PR)])
  pltpu.sync_copy(hist_v.at[pl.ds(0, NB)], sh_hist.at[pl.ds(wid * NB, NB)])
  plsc.subcore_barrier()

  @pl.when(wid == 0)
  def _():
    pltpu.sync_copy(sh_part, stage_v)
    pos_vec = jnp.zeros((L,), jnp.float32)
    npos_vec = jnp.zeros((L,), jnp.int32)
    nneg_vec = jnp.zeros((L,), jnp.int32)
    for t in range(NT):
      pos_vec = pos_vec + lax.bitcast_convert_type(
          stage_v[pl.ds(t * PR, L)], jnp.float32)
      npos_vec = npos_vec + stage_v[pl.ds(t * PR + L, L)]
      nneg_vec = nneg_vec + stage_v[pl.ds(t * PR + 2 * L, L)]
    num_pos = jnp.sum(npos_vec)
    num_neg = jnp.sum(nneg_vec)
    pos_sum = jnp.sum(pos_vec)
    k = jnp.minimum(num_neg, TOTAL_NORM - num_pos)
    k_eff = jnp.maximum(k, 0)

    _merge_staged(sh_hist, merge_v, hist_v, NB)
    b1, k2 = _select_bin(hist_v, NB, k_eff)

    smem_i[0] = k2
    smem_i[1] = b1
    smem_i[2] = k_eff
    smem_f[0] = pos_sum
    _bcast_write(bc_v, sh_bcast, jnp.full((L,), b1, jnp.int32))

  plsc.subcore_barrier()

  pltpu.sync_copy(sh_bcast, part_v)
  pref1 = part_v[pl.ds(0, L)]
  _hist_pass(
      key_v, hist_v,
      lambda key: lax.shift_right_arithmetic(key, 23) == pref1,
      lambda key: lax.shift_right_arithmetic(key, 15) & 0xFF, NB)
  pltpu.sync_copy(hist_v.at[pl.ds(0, NB)], sh_hist.at[pl.ds(wid * NB, NB)])
  plsc.subcore_barrier()

  @pl.when(wid == 0)
  def _():
    k2 = smem_i[0]
    b1 = smem_i[1]
    _merge_staged(sh_hist, merge_v, hist_v, NB)
    b2, k3 = _select_bin(hist_v, NB, k2)
    pref2 = (b1 << 8) | b2
    smem_i[0] = k3
    smem_i[1] = pref2
    _bcast_write(bc_v, sh_bcast, jnp.full((L,), pref2, jnp.int32))

  plsc.subcore_barrier()

  pltpu.sync_copy(sh_bcast, part_v)
  pref2 = part_v[pl.ds(0, L)]
  _hist_pass(
      key_v, hist_v,
      lambda key: lax.shift_right_arithmetic(key, 15) == pref2,
      lambda key: lax.shift_right_arithmetic(key, 7) & 0xFF, NB)
  pltpu.sync_copy(hist_v.at[pl.ds(0, NB)], sh_hist.at[pl.ds(wid * NB, NB)])
  plsc.subcore_barrier()

  @pl.when(wid == 0)
  def _():
    k3 = smem_i[0]
    pref2_s = smem_i[1]
    _merge_staged(sh_hist, merge_v, hist_v, NB)
    b3, k4 = _select_bin(hist_v, NB, k3)
    pref3 = (pref2_s << 8) | b3
    smem_i[0] = k4
    smem_i[1] = pref3
    _bcast_write(bc_v, sh_bcast, jnp.full((L,), pref3, jnp.int32))

  plsc.subcore_barrier()

  pltpu.sync_copy(sh_bcast, part_v)
  pref3 = part_v[pl.ds(0, L)]
  _hist_pass(
      key_v, hist_v,
      lambda key: lax.shift_right_arithmetic(key, 7) == pref3,
      lambda key: key & 0x7F, NB4)
  pltpu.sync_copy(hist_v.at[pl.ds(0, NB4)],
                  sh_hist.at[pl.ds(wid * NB, NB4)])
  plsc.subcore_barrier()

  @pl.when(wid == 0)
  def _():
    k4 = smem_i[0]
    pref3_s = smem_i[1]
    _merge_staged(sh_hist, merge_v, hist_v, NB4)
    b4, k_rem = _select_bin(hist_v, NB4, k4)
    thresh = (pref3_s << 7) | b4
    smem_i[3] = thresh
    smem_i[4] = k_rem
    _bcast_write(bc_v, sh_bcast, jnp.full((L,), thresh, jnp.int32))

  plsc.subcore_barrier()

  pltpu.sync_copy(sh_bcast, part_v)
  t_vec = part_v[pl.ds(0, L)]

  @plsc.parallel_loop(0, NV, unroll=NSLOT, carry=zeros_f)
  def facc(i, acc):
    key = key_v[pl.ds(i * L, L)]
    v = lax.bitcast_convert_type(key, jnp.float32)
    return acc + jnp.where(key > t_vec, v, zeros_f)

  part_v[pl.ds(0, L)] = lax.bitcast_convert_type(facc, jnp.int32)
  pltpu.sync_copy(part_v, sh_fsum.at[pl.ds(wid * PR, PR)])
  plsc.subcore_barrier()

  @pl.when(wid == 0)
  def _():
    pltpu.sync_copy(sh_fsum, stage_v)
    above_vec = jnp.zeros((L,), jnp.float32)
    for t in range(NT):
      above_vec = above_vec + lax.bitcast_convert_type(
          stage_v[pl.ds(t * PR, L)], jnp.float32)
    sum_above = jnp.sum(above_vec)

    pos_sum = smem_f[0]
    k_eff = smem_i[2]
    thresh = smem_i[3]
    k_rem = smem_i[4]

    tie_vec = lax.bitcast_convert_type(
        jnp.full((L,), thresh, jnp.int32), jnp.float32)
    k_rem_f = jnp.full((L,), k_rem, jnp.int32).astype(jnp.float32)
    topk_vec = jnp.full((L,), sum_above, jnp.float32) + k_rem_f * tie_vec
    valid = jnp.full((L,), k_eff, jnp.int32) > 0
    topk_vec = jnp.where(valid, topk_vec, jnp.zeros((L,), jnp.float32))
    loss_vec = (jnp.full((L,), pos_sum, jnp.float32) + topk_vec) * (
        1.0 / TOTAL_NORM)
    tmp_f[...] = loss_vec
    pltpu.sync_copy(tmp_f, out_hbm)


def _rpn_cls_loss_sc(x, tgt):
  run = pl.kernel(
      _sc_body,
      out_type=jax.ShapeDtypeStruct((L,), jnp.float32),
      mesh=_mesh,
      scratch_types=[
          pltpu.VMEM((CHUNK * 2,), jnp.float32),
          pltpu.VMEM((CHUNK,), jnp.int32),
          pltpu.VMEM((CHUNK,), jnp.int32),
          pltpu.VMEM((NSLOT * NB,), jnp.int32),
          pltpu.VMEM((NT * NB,), jnp.int32),
          pltpu.VMEM((PR,), jnp.int32),
          pltpu.VMEM((NT * PR,), jnp.int32),
          pltpu.VMEM((PR,), jnp.int32),
          pltpu.VMEM((L,), jnp.float32),
          pltpu.SMEM((8,), jnp.int32),
          pltpu.SMEM((8,), jnp.float32),
          pltpu.SemaphoreType.DMA,
          pltpu.VMEM_SHARED((NT * NB,), jnp.int32),
          pltpu.VMEM_SHARED((NT * PR,), jnp.int32),
          pltpu.VMEM_SHARED((NT * PR,), jnp.int32),
          pltpu.VMEM_SHARED((PR,), jnp.int32),
      ],
      compiler_params=pltpu.CompilerParams(needs_layout_passes=False),
  )
  return run(x, tgt)


def kernel(input_data, target):
  x = input_data.astype(jnp.float32).reshape(-1)
  tgt = target[0, 0].astype(jnp.int32)
  out = _rpn_cls_loss_sc(x, tgt)
  return out[0]


# speedup vs baseline: 2.7723x; 2.7723x over previous
import jax
import jax.numpy as jnp
from jax import lax
from jax.experimental import pallas as pl
from jax.experimental.pallas import tpu as pltpu
from jax.experimental.pallas import tpu_sc as plsc

L = 16
NT = 16
CHUNK = 6272
NPAD = NT * CHUNK
NV = CHUNK // L
N_IN = 100000
TAIL = N_IN - (NT - 1) * CHUNK
PR = 128

TOTAL_NORM = 60000

NB = 256
NB4 = 128
NSLOT = 4

_mesh = plsc.VectorSubcoreMesh(
    core_axis_name="c", subcore_axis_name="s", num_cores=1)


def _select_bin(hist_ref, nbins, k):
  def total_body(c, acc):
    return acc + jnp.sum(hist_ref[pl.ds(c * L, L)])

  total = lax.fori_loop(0, nbins // L, total_body, jnp.int32(0))
  m = total - k

  lane = lax.iota(jnp.int32, L)

  def body(c, carry):
    run, bsel, psel = carry
    h = hist_ref[pl.ds(c * L, L)]
    p_incl = plsc.cumsum(h) + run
    p_excl = p_incl - h
    m_vec = jnp.full((L,), m, jnp.int32)
    cond = (p_excl <= m_vec) & (m_vec < p_incl)
    zero = jnp.zeros((L,), jnp.int32)
    bsel = bsel + jnp.sum(jnp.where(cond, lane + c * L, zero))
    psel = psel + jnp.sum(jnp.where(cond, p_incl, zero))
    run = run + jnp.sum(h)
    return run, bsel, psel

  _, bsel, psel = lax.fori_loop(
      0, nbins // L, body, (jnp.int32(0), jnp.int32(0), jnp.int32(0)))
  k_rem = psel - m
  return bsel, k_rem


def _zero_hist(hist_ref, nwords):
  zero = jnp.zeros((L,), jnp.int32)

  def body(c, _):
    hist_ref[pl.ds(c * L, L)] = zero
    return 0

  lax.fori_loop(0, nwords // L, body, 0)


def _fold_slots(hist_ref, nbins):
  def body(c, _):
    acc = hist_ref[pl.ds(c * L, L)]
    for u in range(1, NSLOT):
      acc = acc + hist_ref[pl.ds(u * NB + c * L, L)]
    hist_ref[pl.ds(c * L, L)] = acc
    return 0

  lax.fori_loop(0, nbins // L, body, 0)


def _merge_staged(sh_hist, merge_v, hist_v, nbins):
  pltpu.sync_copy(sh_hist, merge_v)

  def body(c, _):
    acc = jnp.zeros((L,), jnp.int32)
    for t in range(NT):
      acc = acc + merge_v[pl.ds(t * NB + c * L, L)]
    hist_v[pl.ds(c * L, L)] = acc
    return 0

  lax.fori_loop(0, nbins // L, body, 0)


def _bcast_write(bc_v, sh_bcast, vec):
  bc_v[pl.ds(0, L)] = vec
  pltpu.sync_copy(bc_v, sh_bcast)


def _sc_body(xt_hbm, tgt_hbm, out_hbm,
             l0_v, l1_v, tgt_v, key_v, ck1_v, ck2_v, hist_v, merge_v,
             part_v, stage_v, bc_v, tmp_f, smem_i, smem_f, sem,
             sh_hist, sh_part, sh_fsum, sh_bcast):
  wid = lax.axis_index("s")
  base = wid * CHUNK

  @pl.when(wid < NT - 1)
  def _():
    c0 = pltpu.make_async_copy(xt_hbm.at[pl.ds(base, CHUNK)], l0_v, sem)
    c1 = pltpu.make_async_copy(xt_hbm.at[pl.ds(N_IN + base, CHUNK)], l1_v, sem)
    c2 = pltpu.make_async_copy(tgt_hbm.at[pl.ds(base, CHUNK)], tgt_v, sem)
    c0.start()
    c1.start()
    c2.start()
    c0.wait()
    c1.wait()
    c2.wait()

  @pl.when(wid == NT - 1)
  def _():
    c0 = pltpu.make_async_copy(
        xt_hbm.at[pl.ds(base, TAIL)], l0_v.at[pl.ds(0, TAIL)], sem)
    c1 = pltpu.make_async_copy(
        xt_hbm.at[pl.ds(N_IN + base, TAIL)], l1_v.at[pl.ds(0, TAIL)], sem)
    c2 = pltpu.make_async_copy(
        tgt_hbm.at[pl.ds(base, TAIL)], tgt_v.at[pl.ds(0, TAIL)], sem)
    c0.start()
    c1.start()
    c2.start()
    c0.wait()
    c1.wait()
    c2.wait()
    pad_t = jnp.full((L,), 2, jnp.int32)

    def fill(c, _):
      tgt_v[pl.ds(TAIL + c * L, L)] = pad_t
      return 0

    lax.fori_loop(0, (CHUNK - TAIL) // L, fill, 0)

  _zero_hist(hist_v, NSLOT * NB)

  ones_i = jnp.ones((L,), jnp.int32)
  zeros_f = jnp.zeros((L,), jnp.float32)
  zeros_i = jnp.zeros((L,), jnp.int32)
  lane = lax.iota(jnp.int32, L)

  @plsc.parallel_loop(0, NV, unroll=NSLOT,
                      carry=(zeros_f, zeros_i, zeros_i))
  def phase_a(i, carry):
    pos_acc, npos_acc, nneg_acc = carry
    sl = pl.ds(i * L, L)
    a0 = l0_v[sl]
    a1 = l1_v[sl]
    t = tgt_v[sl]
    d = a0 - a1
    ad = jnp.abs(d)
    e = jnp.exp(-ad)
    s = e / (2.0 + e)
    s2 = s * s
    p = 1.0 + s2 * (0.33333334 + s2 * (0.2 + s2 * (0.14285715 + s2 * 0.11111111)))
    l1p = 2.0 * s * p
    ce_pos = jnp.maximum(d, 0.0) + l1p
    ce_neg = jnp.maximum(-d, 0.0) + l1p
    is_pos = t == 1
    is_neg = t == 0
    pos_acc = pos_acc + jnp.where(is_pos, ce_pos, zeros_f)
    npos_acc = npos_acc + jnp.where(is_pos, ones_i, zeros_i)
    nneg_acc = nneg_acc + jnp.where(is_neg, ones_i, zeros_i)
    key = jnp.where(is_neg, lax.bitcast_convert_type(ce_neg, jnp.int32), -1)
    key_v[sl] = key
    bins = lax.shift_right_arithmetic(key, 23) + ((i & (NSLOT - 1)) << 8)
    counts, last = plsc.scan_count(bins, mask=is_neg)
    plsc.addupdate_scatter(hist_v, [bins], counts, mask=last)
    return pos_acc, npos_acc, nneg_acc

  pos_acc, npos_acc, nneg_acc = phase_a
  _fold_slots(hist_v, NB)

  part_v[pl.ds(0, L)] = lax.bitcast_convert_type(pos_acc, jnp.int32)
  part_v[pl.ds(L, L)] = npos_acc
  part_v[pl.ds(2 * L, L)] = nneg_acc
  pltpu.sync_copy(part_v, sh_part.at[pl.ds(wid * PR, PR)])
  pltpu.sync_copy(hist_v.at[pl.ds(0, NB)], sh_hist.at[pl.ds(wid * NB, NB)])
  plsc.subcore_barrier()

  @pl.when(wid == 0)
  def _():
    pltpu.sync_copy(sh_part, stage_v)
    pos_vec = jnp.zeros((L,), jnp.float32)
    npos_vec = jnp.zeros((L,), jnp.int32)
    nneg_vec = jnp.zeros((L,), jnp.int32)
    for t in range(NT):
      pos_vec = pos_vec + lax.bitcast_convert_type(
          stage_v[pl.ds(t * PR, L)], jnp.float32)
      npos_vec = npos_vec + stage_v[pl.ds(t * PR + L, L)]
      nneg_vec = nneg_vec + stage_v[pl.ds(t * PR + 2 * L, L)]
    num_pos = jnp.sum(npos_vec)
    num_neg = jnp.sum(nneg_vec)
    pos_sum = jnp.sum(pos_vec)
    k = jnp.minimum(num_neg, TOTAL_NORM - num_pos)
    k_eff = jnp.maximum(k, 0)

    _merge_staged(sh_hist, merge_v, hist_v, NB)
    b1, k2 = _select_bin(hist_v, NB, k_eff)

    smem_i[0] = k2
    smem_i[1] = b1
    smem_i[2] = k_eff
    smem_f[0] = pos_sum
    _bcast_write(bc_v, sh_bcast, jnp.full((L,), b1, jnp.int32))

  plsc.subcore_barrier()

  pltpu.sync_copy(sh_bcast, part_v)
  pref1 = part_v[pl.ds(0, L)]
  _zero_hist(hist_v, NSLOT * NB)

  @plsc.parallel_loop(0, NV, unroll=NSLOT, carry=jnp.int32(0))
  def n1(i, off):
    key = key_v[pl.ds(i * L, L)]
    match = lax.shift_right_arithmetic(key, 23) == pref1
    bins = (lax.shift_right_arithmetic(key, 15) & 0xFF) + (
        (i & (NSLOT - 1)) << 8)
    counts, last = plsc.scan_count(bins, mask=match)
    plsc.addupdate_scatter(hist_v, [bins], counts, mask=last)
    plsc.store_compressed(ck1_v.at[pl.ds(off, L)], key, mask=match)
    return off + jnp.sum(jnp.where(match, ones_i, zeros_i))

  _fold_slots(hist_v, NB)
  pltpu.sync_copy(hist_v.at[pl.ds(0, NB)], sh_hist.at[pl.ds(wid * NB, NB)])
  plsc.subcore_barrier()

  @pl.when(wid == 0)
  def _():
    k2 = smem_i[0]
    b1 = smem_i[1]
    _merge_staged(sh_hist, merge_v, hist_v, NB)
    b2, k3 = _select_bin(hist_v, NB, k2)
    pref2 = (b1 << 8) | b2
    smem_i[0] = k3
    smem_i[1] = pref2
    _bcast_write(bc_v, sh_bcast, jnp.full((L,), pref2, jnp.int32))

  plsc.subcore_barrier()

  pltpu.sync_copy(sh_bcast, part_v)
  pref2 = part_v[pl.ds(0, L)]
  _zero_hist(hist_v, NSLOT * NB)
  n1_vec = jnp.full((L,), n1, jnp.int32)
  trip1 = (n1 + L - 1) // L

  @plsc.parallel_loop(0, trip1, unroll=NSLOT, carry=jnp.int32(0))
  def n2(c, off):
    key = ck1_v[pl.ds(c * L, L)]
    valid = (c * L + lane) < n1_vec
    match = valid & (lax.shift_right_arithmetic(key, 15) == pref2)
    bins = (lax.shift_right_arithmetic(key, 7) & 0xFF) + (
        (c & (NSLOT - 1)) << 8)
    counts, last = plsc.scan_count(bins, mask=match)
    plsc.addupdate_scatter(hist_v, [bins], counts, mask=last)
    plsc.store_compressed(ck2_v.at[pl.ds(off, L)], key, mask=match)
    return off + jnp.sum(jnp.where(match, ones_i, zeros_i))

  _fold_slots(hist_v, NB)
  pltpu.sync_copy(hist_v.at[pl.ds(0, NB)], sh_hist.at[pl.ds(wid * NB, NB)])
  plsc.subcore_barrier()

  @pl.when(wid == 0)
  def _():
    k3 = smem_i[0]
    pref2_s = smem_i[1]
    _merge_staged(sh_hist, merge_v, hist_v, NB)
    b3, k4 = _select_bin(hist_v, NB, k3)
    pref3 = (pref2_s << 8) | b3
    smem_i[0] = k4
    smem_i[1] = pref3
    _bcast_write(bc_v, sh_bcast, jnp.full((L,), pref3, jnp.int32))

  plsc.subcore_barrier()

  pltpu.sync_copy(sh_bcast, part_v)
  pref3 = part_v[pl.ds(0, L)]
  _zero_hist(hist_v, NSLOT * NB)
  n2_vec = jnp.full((L,), n2, jnp.int32)
  trip2 = (n2 + L - 1) // L

  @plsc.parallel_loop(0, trip2, unroll=NSLOT)
  def _(c):
    key = ck2_v[pl.ds(c * L, L)]
    valid = (c * L + lane) < n2_vec
    match = valid & (lax.shift_right_arithmetic(key, 7) == pref3)
    bins = (key & 0x7F) + ((c & (NSLOT - 1)) << 8)
    counts, last = plsc.scan_count(bins, mask=match)
    plsc.addupdate_scatter(hist_v, [bins], counts, mask=last)

  _fold_slots(hist_v, NB4)
  pltpu.sync_copy(hist_v.at[pl.ds(0, NB4)],
                  sh_hist.at[pl.ds(wid * NB, NB4)])
  plsc.subcore_barrier()

  @pl.when(wid == 0)
  def _():
    k4 = smem_i[0]
    pref3_s = smem_i[1]
    _merge_staged(sh_hist, merge_v, hist_v, NB4)
    b4, k_rem = _select_bin(hist_v, NB4, k4)
    thresh = (pref3_s << 7) | b4
    smem_i[3] = thresh
    smem_i[4] = k_rem
    _bcast_write(bc_v, sh_bcast, jnp.full((L,), thresh, jnp.int32))

  plsc.subcore_barrier()

  pltpu.sync_copy(sh_bcast, part_v)
  t_vec = part_v[pl.ds(0, L)]

  @plsc.parallel_loop(0, NV, unroll=NSLOT, carry=zeros_f)
  def facc(i, acc):
    key = key_v[pl.ds(i * L, L)]
    v = lax.bitcast_convert_type(key, jnp.float32)
    return acc + jnp.where(key > t_vec, v, zeros_f)

  part_v[pl.ds(0, L)] = lax.bitcast_convert_type(facc, jnp.int32)
  pltpu.sync_copy(part_v, sh_fsum.at[pl.ds(wid * PR, PR)])
  plsc.subcore_barrier()

  @pl.when(wid == 0)
  def _():
    pltpu.sync_copy(sh_fsum, stage_v)
    above_vec = jnp.zeros((L,), jnp.float32)
    for t in range(NT):
      above_vec = above_vec + lax.bitcast_convert_type(
          stage_v[pl.ds(t * PR, L)], jnp.float32)
    sum_above = jnp.sum(above_vec)

    pos_sum = smem_f[0]
    k_eff = smem_i[2]
    thresh = smem_i[3]
    k_rem = smem_i[4]

    tie_vec = lax.bitcast_convert_type(
        jnp.full((L,), thresh, jnp.int32), jnp.float32)
    k_rem_f = jnp.full((L,), k_rem, jnp.int32).astype(jnp.float32)
    topk_vec = jnp.full((L,), sum_above, jnp.float32) + k_rem_f * tie_vec
    valid = jnp.full((L,), k_eff, jnp.int32) > 0
    topk_vec = jnp.where(valid, topk_vec, jnp.zeros((L,), jnp.float32))
    loss_vec = (jnp.full((L,), pos_sum, jnp.float32) + topk_vec) * (
        1.0 / TOTAL_NORM)
    tmp_f[...] = loss_vec
    pltpu.sync_copy(tmp_f, out_hbm)


def _rpn_cls_loss_sc(xt, tgt):
  run = pl.kernel(
      _sc_body,
      out_type=jax.ShapeDtypeStruct((L,), jnp.float32),
      mesh=_mesh,
      scratch_types=[
          pltpu.VMEM((CHUNK,), jnp.float32),
          pltpu.VMEM((CHUNK,), jnp.float32),
          pltpu.VMEM((CHUNK,), jnp.int32),
          pltpu.VMEM((CHUNK,), jnp.int32),
          pltpu.VMEM((CHUNK + L,), jnp.int32),
          pltpu.VMEM((CHUNK + L,), jnp.int32),
          pltpu.VMEM((NSLOT * NB,), jnp.int32),
          pltpu.VMEM((NT * NB,), jnp.int32),
          pltpu.VMEM((PR,), jnp.int32),
          pltpu.VMEM((NT * PR,), jnp.int32),
          pltpu.VMEM((PR,), jnp.int32),
          pltpu.VMEM((L,), jnp.float32),
          pltpu.SMEM((8,), jnp.int32),
          pltpu.SMEM((8,), jnp.float32),
          pltpu.SemaphoreType.DMA,
          pltpu.VMEM_SHARED((NT * NB,), jnp.int32),
          pltpu.VMEM_SHARED((NT * PR,), jnp.int32),
          pltpu.VMEM_SHARED((NT * PR,), jnp.int32),
          pltpu.VMEM_SHARED((PR,), jnp.int32),
      ],
      compiler_params=pltpu.CompilerParams(needs_layout_passes=False),
  )
  return run(xt, tgt)


def kernel(input_data, target):
  xt = input_data[0].astype(jnp.float32).T.reshape(-1)
  tgt = target[0, 0].astype(jnp.int32)
  out = _rpn_cls_loss_sc(xt, tgt)
  return out[0]


# speedup vs baseline: 2.7891x; 1.0061x over previous
import jax
import jax.numpy as jnp
from jax import lax
from jax.experimental import pallas as pl
from jax.experimental.pallas import tpu as pltpu
from jax.experimental.pallas import tpu_sc as plsc

L = 16
NT = 16
CHUNK = 6272
NPAD = NT * CHUNK
NV = CHUNK // L
N_IN = 100000
TAIL = N_IN - (NT - 1) * CHUNK
PR = 128

TOTAL_NORM = 60000

NB = 256
NB4 = 128
NSLOT = 4

_mesh = plsc.VectorSubcoreMesh(
    core_axis_name="c", subcore_axis_name="s", num_cores=1)


def _select_bin(hist_ref, nbins, k, total):
  m = total - k

  lane = lax.iota(jnp.int32, L)

  def body(c, carry):
    run, bsel, psel, hsel = carry
    h = hist_ref[pl.ds(c * L, L)]
    p_incl = plsc.cumsum(h) + run
    p_excl = p_incl - h
    m_vec = jnp.full((L,), m, jnp.int32)
    cond = (p_excl <= m_vec) & (m_vec < p_incl)
    zero = jnp.zeros((L,), jnp.int32)
    bsel = bsel + jnp.sum(jnp.where(cond, lane + c * L, zero))
    psel = psel + jnp.sum(jnp.where(cond, p_incl, zero))
    hsel = hsel + jnp.sum(jnp.where(cond, h, zero))
    run = run + jnp.sum(h)
    return run, bsel, psel, hsel

  _, bsel, psel, hsel = lax.fori_loop(
      0, nbins // L, body,
      (jnp.int32(0), jnp.int32(0), jnp.int32(0), jnp.int32(0)))
  k_rem = psel - m
  return bsel, k_rem, hsel


def _zero_hist(hist_ref, nwords):
  zero = jnp.zeros((L,), jnp.int32)

  def body(c, _):
    hist_ref[pl.ds(c * L, L)] = zero
    return 0

  lax.fori_loop(0, nwords // L, body, 0)


def _fold_slots(hist_ref, nbins):
  def body(c, _):
    acc = hist_ref[pl.ds(c * L, L)]
    for u in range(1, NSLOT):
      acc = acc + hist_ref[pl.ds(u * NB + c * L, L)]
    hist_ref[pl.ds(c * L, L)] = acc
    return 0

  lax.fori_loop(0, nbins // L, body, 0)


def _merge_staged(sh_hist, merge_v, hist_v, nbins):
  pltpu.sync_copy(sh_hist, merge_v)

  def body(c, _):
    acc = jnp.zeros((L,), jnp.int32)
    for t in range(NT):
      acc = acc + merge_v[pl.ds(t * NB + c * L, L)]
    hist_v[pl.ds(c * L, L)] = acc
    return 0

  lax.fori_loop(0, nbins // L, body, 0)


def _bcast_write(bc_v, sh_bcast, vec):
  bc_v[pl.ds(0, L)] = vec
  pltpu.sync_copy(bc_v, sh_bcast)


def _sc_body(xt_hbm, tgt_hbm, out_hbm,
             l0_v, l1_v, tgt_v, key_v, ck1_v, ck2_v, hist_v, merge_v,
             part_v, stage_v, bc_v, tmp_f, smem_i, smem_f, sem,
             sh_hist, sh_part, sh_fsum, sh_bcast):
  wid = lax.axis_index("s")
  base = wid * CHUNK

  @pl.when(wid < NT - 1)
  def _():
    c0 = pltpu.make_async_copy(xt_hbm.at[pl.ds(base, CHUNK)], l0_v, sem)
    c1 = pltpu.make_async_copy(xt_hbm.at[pl.ds(N_IN + base, CHUNK)], l1_v, sem)
    c2 = pltpu.make_async_copy(tgt_hbm.at[pl.ds(base, CHUNK)], tgt_v, sem)
    c0.start()
    c1.start()
    c2.start()
    c0.wait()
    c1.wait()
    c2.wait()

  @pl.when(wid == NT - 1)
  def _():
    c0 = pltpu.make_async_copy(
        xt_hbm.at[pl.ds(base, TAIL)], l0_v.at[pl.ds(0, TAIL)], sem)
    c1 = pltpu.make_async_copy(
        xt_hbm.at[pl.ds(N_IN + base, TAIL)], l1_v.at[pl.ds(0, TAIL)], sem)
    c2 = pltpu.make_async_copy(
        tgt_hbm.at[pl.ds(base, TAIL)], tgt_v.at[pl.ds(0, TAIL)], sem)
    c0.start()
    c1.start()
    c2.start()
    c0.wait()
    c1.wait()
    c2.wait()
    pad_t = jnp.full((L,), 2, jnp.int32)

    def fill(c, _):
      tgt_v[pl.ds(TAIL + c * L, L)] = pad_t
      return 0

    lax.fori_loop(0, (CHUNK - TAIL) // L, fill, 0)

  _zero_hist(hist_v, NSLOT * NB)

  ones_i = jnp.ones((L,), jnp.int32)
  zeros_f = jnp.zeros((L,), jnp.float32)
  zeros_i = jnp.zeros((L,), jnp.int32)
  lane = lax.iota(jnp.int32, L)

  @plsc.parallel_loop(0, NV, unroll=NSLOT,
                      carry=(zeros_f, zeros_i, zeros_i))
  def phase_a(i, carry):
    pos_acc, npos_acc, nneg_acc = carry
    sl = pl.ds(i * L, L)
    a0 = l0_v[sl]
    a1 = l1_v[sl]
    t = tgt_v[sl]
    d = a0 - a1
    ad = jnp.abs(d)
    e = jnp.exp(-ad)
    s = e / (2.0 + e)
    s2 = s * s
    p = 1.0 + s2 * (0.33333334 + s2 * (0.2 + s2 * (0.14285715 + s2 * 0.11111111)))
    l1p = 2.0 * s * p
    ce_pos = jnp.maximum(d, 0.0) + l1p
    ce_neg = jnp.maximum(-d, 0.0) + l1p
    is_pos = t == 1
    is_neg = t == 0
    pos_acc = pos_acc + jnp.where(is_pos, ce_pos, zeros_f)
    npos_acc = npos_acc + jnp.where(is_pos, ones_i, zeros_i)
    nneg_acc = nneg_acc + jnp.where(is_neg, ones_i, zeros_i)
    key = jnp.where(is_neg, lax.bitcast_convert_type(ce_neg, jnp.int32), -1)
    key_v[sl] = key
    bins = lax.shift_right_arithmetic(key, 23) + ((i & (NSLOT - 1)) << 8)
    counts, last = plsc.scan_count(bins, mask=is_neg)
    plsc.addupdate_scatter(hist_v, [bins], counts, mask=last)
    return pos_acc, npos_acc, nneg_acc

  pos_acc, npos_acc, nneg_acc = phase_a
  _fold_slots(hist_v, NB)

  part_v[pl.ds(0, L)] = lax.bitcast_convert_type(pos_acc, jnp.int32)
  part_v[pl.ds(L, L)] = npos_acc
  part_v[pl.ds(2 * L, L)] = nneg_acc
  pltpu.sync_copy(part_v, sh_part.at[pl.ds(wid * PR, PR)])
  pltpu.sync_copy(hist_v.at[pl.ds(0, NB)], sh_hist.at[pl.ds(wid * NB, NB)])
  plsc.subcore_barrier()

  @pl.when(wid == 0)
  def _():
    pltpu.sync_copy(sh_part, stage_v)
    pos_vec = jnp.zeros((L,), jnp.float32)
    npos_vec = jnp.zeros((L,), jnp.int32)
    nneg_vec = jnp.zeros((L,), jnp.int32)
    for t in range(NT):
      pos_vec = pos_vec + lax.bitcast_convert_type(
          stage_v[pl.ds(t * PR, L)], jnp.float32)
      npos_vec = npos_vec + stage_v[pl.ds(t * PR + L, L)]
      nneg_vec = nneg_vec + stage_v[pl.ds(t * PR + 2 * L, L)]
    num_pos = jnp.sum(npos_vec)
    num_neg = jnp.sum(nneg_vec)
    pos_sum = jnp.sum(pos_vec)
    k = jnp.minimum(num_neg, TOTAL_NORM - num_pos)
    k_eff = jnp.maximum(k, 0)

    _merge_staged(sh_hist, merge_v, hist_v, NB)
    b1, k2, h1 = _select_bin(hist_v, NB, k_eff, num_neg)

    smem_i[0] = k2
    smem_i[1] = b1
    smem_i[2] = k_eff
    smem_i[5] = h1
    smem_f[0] = pos_sum
    _bcast_write(bc_v, sh_bcast, jnp.full((L,), b1, jnp.int32))

  plsc.subcore_barrier()

  pltpu.sync_copy(sh_bcast, part_v)
  pref1 = part_v[pl.ds(0, L)]
  _zero_hist(hist_v, NSLOT * NB)

  @plsc.parallel_loop(0, NV, unroll=NSLOT, carry=jnp.int32(0))
  def n1(i, off):
    key = key_v[pl.ds(i * L, L)]
    match = lax.shift_right_arithmetic(key, 23) == pref1
    bins = (lax.shift_right_arithmetic(key, 15) & 0xFF) + (
        (i & (NSLOT - 1)) << 8)
    counts, last = plsc.scan_count(bins, mask=match)
    plsc.addupdate_scatter(hist_v, [bins], counts, mask=last)
    plsc.store_compressed(ck1_v.at[pl.ds(off, L)], key, mask=match)
    return off + jnp.sum(jnp.where(match, ones_i, zeros_i))

  _fold_slots(hist_v, NB)
  pltpu.sync_copy(hist_v.at[pl.ds(0, NB)], sh_hist.at[pl.ds(wid * NB, NB)])
  plsc.subcore_barrier()

  @pl.when(wid == 0)
  def _():
    k2 = smem_i[0]
    b1 = smem_i[1]
    _merge_staged(sh_hist, merge_v, hist_v, NB)
    b2, k3, h2 = _select_bin(hist_v, NB, k2, smem_i[5])
    pref2 = (b1 << 8) | b2
    smem_i[0] = k3
    smem_i[1] = pref2
    smem_i[5] = h2
    _bcast_write(bc_v, sh_bcast, jnp.full((L,), pref2, jnp.int32))

  plsc.subcore_barrier()

  pltpu.sync_copy(sh_bcast, part_v)
  pref2 = part_v[pl.ds(0, L)]
  _zero_hist(hist_v, NSLOT * NB)
  n1_vec = jnp.full((L,), n1, jnp.int32)
  trip1 = (n1 + L - 1) // L

  @plsc.parallel_loop(0, trip1, unroll=NSLOT, carry=jnp.int32(0))
  def n2(c, off):
    key = ck1_v[pl.ds(c * L, L)]
    valid = (c * L + lane) < n1_vec
    match = valid & (lax.shift_right_arithmetic(key, 15) == pref2)
    bins = (lax.shift_right_arithmetic(key, 7) & 0xFF) + (
        (c & (NSLOT - 1)) << 8)
    counts, last = plsc.scan_count(bins, mask=match)
    plsc.addupdate_scatter(hist_v, [bins], counts, mask=last)
    plsc.store_compressed(ck2_v.at[pl.ds(off, L)], key, mask=match)
    return off + jnp.sum(jnp.where(match, ones_i, zeros_i))

  _fold_slots(hist_v, NB)
  pltpu.sync_copy(hist_v.at[pl.ds(0, NB)], sh_hist.at[pl.ds(wid * NB, NB)])
  plsc.subcore_barrier()

  @pl.when(wid == 0)
  def _():
    k3 = smem_i[0]
    pref2_s = smem_i[1]
    _merge_staged(sh_hist, merge_v, hist_v, NB)
    b3, k4, h3 = _select_bin(hist_v, NB, k3, smem_i[5])
    pref3 = (pref2_s << 8) | b3
    smem_i[0] = k4
    smem_i[1] = pref3
    smem_i[5] = h3
    _bcast_write(bc_v, sh_bcast, jnp.full((L,), pref3, jnp.int32))

  plsc.subcore_barrier()

  pltpu.sync_copy(sh_bcast, part_v)
  pref3 = part_v[pl.ds(0, L)]
  _zero_hist(hist_v, NSLOT * NB)
  n2_vec = jnp.full((L,), n2, jnp.int32)
  trip2 = (n2 + L - 1) // L

  @plsc.parallel_loop(0, trip2, unroll=NSLOT)
  def _(c):
    key = ck2_v[pl.ds(c * L, L)]
    valid = (c * L + lane) < n2_vec
    match = valid & (lax.shift_right_arithmetic(key, 7) == pref3)
    bins = (key & 0x7F) + ((c & (NSLOT - 1)) << 8)
    counts, last = plsc.scan_count(bins, mask=match)
    plsc.addupdate_scatter(hist_v, [bins], counts, mask=last)

  _fold_slots(hist_v, NB4)
  pltpu.sync_copy(hist_v.at[pl.ds(0, NB4)],
                  sh_hist.at[pl.ds(wid * NB, NB4)])
  plsc.subcore_barrier()

  @pl.when(wid == 0)
  def _():
    k4 = smem_i[0]
    pref3_s = smem_i[1]
    _merge_staged(sh_hist, merge_v, hist_v, NB4)
    b4, k_rem, _ = _select_bin(hist_v, NB4, k4, smem_i[5])
    thresh = (pref3_s << 7) | b4
    smem_i[3] = thresh
    smem_i[4] = k_rem
    _bcast_write(bc_v, sh_bcast, jnp.full((L,), thresh, jnp.int32))

  plsc.subcore_barrier()

  pltpu.sync_copy(sh_bcast, part_v)
  t_vec = part_v[pl.ds(0, L)]

  @plsc.parallel_loop(0, NV, unroll=NSLOT, carry=zeros_f)
  def facc(i, acc):
    key = key_v[pl.ds(i * L, L)]
    v = lax.bitcast_convert_type(key, jnp.float32)
    return acc + jnp.where(key > t_vec, v, zeros_f)

  part_v[pl.ds(0, L)] = lax.bitcast_convert_type(facc, jnp.int32)
  pltpu.sync_copy(part_v, sh_fsum.at[pl.ds(wid * PR, PR)])
  plsc.subcore_barrier()

  @pl.when(wid == 0)
  def _():
    pltpu.sync_copy(sh_fsum, stage_v)
    above_vec = jnp.zeros((L,), jnp.float32)
    for t in range(NT):
      above_vec = above_vec + lax.bitcast_convert_type(
          stage_v[pl.ds(t * PR, L)], jnp.float32)
    sum_above = jnp.sum(above_vec)

    pos_sum = smem_f[0]
    k_eff = smem_i[2]
    thresh = smem_i[3]
    k_rem = smem_i[4]

    tie_vec = lax.bitcast_convert_type(
        jnp.full((L,), thresh, jnp.int32), jnp.float32)
    k_rem_f = jnp.full((L,), k_rem, jnp.int32).astype(jnp.float32)
    topk_vec = jnp.full((L,), sum_above, jnp.float32) + k_rem_f * tie_vec
    valid = jnp.full((L,), k_eff, jnp.int32) > 0
    topk_vec = jnp.where(valid, topk_vec, jnp.zeros((L,), jnp.float32))
    loss_vec = (jnp.full((L,), pos_sum, jnp.float32) + topk_vec) * (
        1.0 / TOTAL_NORM)
    tmp_f[...] = loss_vec
    pltpu.sync_copy(tmp_f, out_hbm)


def _rpn_cls_loss_sc(xt, tgt):
  run = pl.kernel(
      _sc_body,
      out_type=jax.ShapeDtypeStruct((L,), jnp.float32),
      mesh=_mesh,
      scratch_types=[
          pltpu.VMEM((CHUNK,), jnp.float32),
          pltpu.VMEM((CHUNK,), jnp.float32),
          pltpu.VMEM((CHUNK,), jnp.int32),
          pltpu.VMEM((CHUNK,), jnp.int32),
          pltpu.VMEM((CHUNK + L,), jnp.int32),
          pltpu.VMEM((CHUNK + L,), jnp.int32),
          pltpu.VMEM((NSLOT * NB,), jnp.int32),
          pltpu.VMEM((NT * NB,), jnp.int32),
          pltpu.VMEM((PR,), jnp.int32),
          pltpu.VMEM((NT * PR,), jnp.int32),
          pltpu.VMEM((PR,), jnp.int32),
          pltpu.VMEM((L,), jnp.float32),
          pltpu.SMEM((8,), jnp.int32),
          pltpu.SMEM((8,), jnp.float32),
          pltpu.SemaphoreType.DMA,
          pltpu.VMEM_SHARED((NT * NB,), jnp.int32),
          pltpu.VMEM_SHARED((NT * PR,), jnp.int32),
          pltpu.VMEM_SHARED((NT * PR,), jnp.int32),
          pltpu.VMEM_SHARED((PR,), jnp.int32),
      ],
      compiler_params=pltpu.CompilerParams(needs_layout_passes=False),
  )
  return run(xt, tgt)


def kernel(input_data, target):
  xt = input_data[0].astype(jnp.float32).T.reshape(-1)
  tgt = target[0, 0].astype(jnp.int32)
  out = _rpn_cls_loss_sc(xt, tgt)
  return out[0]
